# baseline (speedup 1.0000x reference)
import jax
import jax.numpy as jnp
from jax import lax
from jax.experimental import pallas as pl
from jax.experimental.pallas import tpu as pltpu
from jax.experimental.pallas import tpu_sc as plsc

N_ROWS = 1000000
D = 32
B = 16384

_NC = 2
_NS = 16
_NW = _NC * _NS
_BPW = B // _NW


def _gather_body(table_hbm, idx_hbm, out_hbm, idx_v, rows_v, sem):
    wid = lax.axis_index("s") * _NC + lax.axis_index("c")
    base = wid * _BPW
    pltpu.sync_copy(idx_hbm.at[pl.ds(base, _BPW)], idx_v)
    pltpu.async_copy(table_hbm.at[idx_v], rows_v, sem).wait()
    pltpu.sync_copy(rows_v, out_hbm.at[pl.ds(base, _BPW)])


def kernel(input, weight):
    idx = input.astype(jnp.int32)
    mesh = plsc.VectorSubcoreMesh(core_axis_name="c", subcore_axis_name="s")
    f = pl.kernel(
        _gather_body,
        mesh=mesh,
        out_type=jax.ShapeDtypeStruct((B, D), jnp.float32),
        scratch_types=[
            pltpu.VMEM((_BPW,), jnp.int32),
            pltpu.VMEM((_BPW, D), jnp.float32),
            pltpu.SemaphoreType.DMA,
        ],
        compiler_params=pltpu.CompilerParams(use_tc_tiling_on_sc=False),
    )
    return f(weight, idx)


# speedup vs baseline: 4.6230x; 4.6230x over previous
import jax
import jax.numpy as jnp
from jax import lax
from jax.experimental import pallas as pl
from jax.experimental.pallas import tpu as pltpu
from jax.experimental.pallas import tpu_sc as plsc

N_ROWS = 1000000
D = 32
B = 16384

_NC = 2
_NS = 16
_NW = _NC * _NS
_BPW = B // _NW
_CH = 8
_NCH = _BPW // _CH


def _gather_body(wt_hbm, idx_hbm, out_hbm, idx_v, idx_sm, blk0, blk1, cols_v, sem):
    wid = lax.axis_index("s") * _NC + lax.axis_index("c")
    base = wid * _BPW
    pltpu.sync_copy(idx_hbm.at[pl.ds(base, _BPW)], idx_v)

    lanes = lax.iota(jnp.int32, 16)

    def stage(k):
        v = idx_v[pl.ds(k * 16, 16)]
        for j in range(16):
            idx_sm[k * 16 + j] = v[j]

    pl.loop(0, _BPW // 16)(stage)

    def fetch(ch, blk):
        ch0 = ch * _CH
        for i in range(_CH):
            x = idx_sm[ch0 + i]
            c = (x >> 7) * 128
            pltpu.async_copy(wt_hbm.at[:, pl.ds(c, 128)], blk.at[i], sem)

    def drain(blk):
        pltpu.make_async_copy(wt_hbm.at[:, pl.ds(0, 128 * _CH)], blk, sem).wait()

    def extract(ch, blk):
        ch0 = ch * _CH
        for i in range(_CH):
            x = idx_sm[ch0 + i]
            lvec = jnp.full((16,), x & 127, dtype=jnp.int32)
            ivec = jnp.full((16,), i, jnp.int32)
            ovec = jnp.full((16,), ch0 + i, jnp.int32)
            top = plsc.load_gather(blk, [ivec, lanes, lvec])
            bot = plsc.load_gather(blk, [ivec, lanes + 16, lvec])
            plsc.store_scatter(cols_v, [lanes, ovec], top)
            plsc.store_scatter(cols_v, [lanes + 16, ovec], bot)

    fetch(0, blk0)

    def chunk_pair(ch):
        fetch(ch + 1, blk1)
        drain(blk0)
        extract(ch, blk0)

        @pl.when(ch + 2 < _NCH)
        def _():
            fetch(ch + 2, blk0)

        drain(blk1)
        extract(ch + 1, blk1)

    pl.loop(0, _NCH, step=2)(chunk_pair)
    pltpu.sync_copy(cols_v, out_hbm.at[:, pl.ds(base, _BPW)])


def kernel(input, weight):
    idx = input.astype(jnp.int32)
    mesh = plsc.VectorSubcoreMesh(core_axis_name="c", subcore_axis_name="s")
    f = pl.kernel(
        _gather_body,
        mesh=mesh,
        out_type=jax.ShapeDtypeStruct((D, B), jnp.float32),
        scratch_types=[
            pltpu.VMEM((_BPW,), jnp.int32),
            pltpu.SMEM((_BPW,), jnp.int32),
            pltpu.VMEM((_CH, D, 128), jnp.float32),
            pltpu.VMEM((_CH, D, 128), jnp.float32),
            pltpu.VMEM((D, _BPW), jnp.float32),
            pltpu.SemaphoreType.DMA,
        ],
        compiler_params=pltpu.CompilerParams(
            use_tc_tiling_on_sc=True, needs_layout_passes=False
        ),
    )
    return f(weight.T, idx).T
